# baseline (speedup 1.0000x reference)
import functools
import math

import jax
import jax.numpy as jnp
from jax.experimental import pallas as pl

_PADDING_IDX = 0
_SMOOTHING = 0.1
_CONFIDENCE = 1.0 - _SMOOTHING


def _loss_kernel(t_ref, x_ref, o_ref, *, col_block, eps, row_const):
    j = pl.program_id(1)
    first = (pl.program_id(0) == 0) & (j == 0)

    x = x_ref[...]
    t = t_ref[0, 0, :]
    m = (t != _PADDING_IDX).astype(jnp.float32)

    cols = j * col_block + jax.lax.broadcasted_iota(jnp.int32, x.shape, 1)
    sel = jnp.where(cols == t[:, None], x, 0.0)
    s = jnp.sum(x, axis=1)
    gs = jnp.sum(sel, axis=1)
    partial = jnp.sum(m * ((eps - _CONFIDENCE) * gs - eps * s))

    @pl.when(first)
    def _():
        o_ref[...] = jnp.zeros_like(o_ref)

    @pl.when(j == 0)
    def _():
        head = eps * jnp.sum(x[:, 0] * m) + row_const * jnp.sum(m)
        o_ref[...] += jnp.full((1, 1), head, dtype=jnp.float32)

    o_ref[...] += jnp.full((1, 1), partial, dtype=jnp.float32)


def kernel(x, target):
    n, v = x.shape
    row_block = 128
    col_block = 16000
    nr = n // row_block
    nc = v // col_block

    eps = _SMOOTHING / (v - 2)
    row_const = _SMOOTHING * math.log(eps) + _CONFIDENCE * math.log(_CONFIDENCE)

    t32 = target.astype(jnp.int32)
    t3 = t32.reshape(nr, 1, row_block)

    out = pl.pallas_call(
        functools.partial(
            _loss_kernel, col_block=col_block, eps=eps, row_const=row_const
        ),
        grid=(nr, nc),
        in_specs=[
            pl.BlockSpec((1, 1, row_block), lambda i, j: (i, 0, 0)),
            pl.BlockSpec((row_block, col_block), lambda i, j: (i, j)),
        ],
        out_specs=pl.BlockSpec((1, 1), lambda i, j: (0, 0)),
        out_shape=jax.ShapeDtypeStruct((1, 1), jnp.float32),
    )(t3, x)
    return out[0, 0]


# speedup vs baseline: 1.1047x; 1.1047x over previous
import functools
import math

import jax
import jax.numpy as jnp
from jax.experimental import pallas as pl

_PADDING_IDX = 0
_SMOOTHING = 0.1
_CONFIDENCE = 1.0 - _SMOOTHING


def _loss_kernel(t_ref, x_ref, o_ref, *, col_block, eps, row_const):
    j = pl.program_id(1)
    first = (pl.program_id(0) == 0) & (j == 0)

    x = x_ref[...]
    t = t_ref[0, 0, :]
    m = (t != _PADDING_IDX).astype(jnp.float32)

    cols = j * col_block + jax.lax.broadcasted_iota(jnp.int32, x.shape, 1)
    sel = jnp.where(cols == t[:, None], x, 0.0)
    s = jnp.sum(x, axis=1)
    gs = jnp.sum(sel, axis=1)
    partial = jnp.sum(m * ((eps - _CONFIDENCE) * gs - eps * s))

    @pl.when(first)
    def _():
        o_ref[...] = jnp.zeros_like(o_ref)

    @pl.when(j == 0)
    def _():
        head = eps * jnp.sum(x[:, 0] * m) + row_const * jnp.sum(m)
        o_ref[...] += jnp.full((1, 1), head, dtype=jnp.float32)

    o_ref[...] += jnp.full((1, 1), partial, dtype=jnp.float32)


def kernel(x, target):
    n, v = x.shape
    row_block = 256
    col_block = 16000
    nr = n // row_block
    nc = v // col_block

    eps = _SMOOTHING / (v - 2)
    row_const = _SMOOTHING * math.log(eps) + _CONFIDENCE * math.log(_CONFIDENCE)

    t32 = target.astype(jnp.int32)
    t3 = t32.reshape(nr, 1, row_block)

    out = pl.pallas_call(
        functools.partial(
            _loss_kernel, col_block=col_block, eps=eps, row_const=row_const
        ),
        grid=(nr, nc),
        in_specs=[
            pl.BlockSpec((1, 1, row_block), lambda i, j: (i, 0, 0)),
            pl.BlockSpec((row_block, col_block), lambda i, j: (i, j)),
        ],
        out_specs=pl.BlockSpec((1, 1), lambda i, j: (0, 0)),
        out_shape=jax.ShapeDtypeStruct((1, 1), jnp.float32),
    )(t3, x)
    return out[0, 0]
